# Initial kernel scaffold; baseline (speedup 1.0000x reference)
import jax
import jax.numpy as jnp
from jax.experimental import pallas as pl


def kernel(idx, log_probs):
    raise NotImplementedError("write your pallas kernel here")


# speedup vs baseline: 1.2021x; 1.2021x over previous
import functools

import jax
import jax.numpy as jnp
from jax import lax
from jax.experimental import pallas as pl
from jax.experimental.pallas import tpu as pltpu
from jax.experimental.pallas import tpu_sc as plsc

_NC = 2
_NS = 16
_NW = _NC * _NS


_TP = 56


@functools.cache
def _build(B, T, V, D, dtype):
    BPW = B // _NW

    mesh = plsc.VectorSubcoreMesh(
        core_axis_name="c", subcore_axis_name="s",
        num_cores=_NC, num_subcores=_NS)

    @functools.partial(
        pl.kernel,
        mesh=mesh,
        out_type=jax.ShapeDtypeStruct((B * T, D), dtype),
        compiler_params=pltpu.CompilerParams(use_tc_tiling_on_sc=False),
        scratch_types=[
            pltpu.VMEM((BPW * _TP,), jnp.int32),
            pltpu.VMEM((T, D), dtype),
            pltpu.VMEM((T, D), dtype),
            pltpu.SemaphoreType.DMA,
            pltpu.SemaphoreType.DMA,
            pltpu.SemaphoreType.DMA,
            pltpu.SemaphoreType.DMA,
        ],
    )
    def run(idx_hbm, tab_hbm, zrow_hbm, out_hbm,
            idxv, buf0, buf1, g0, g1, w0, w1):
        wid = lax.axis_index("s") * _NC + lax.axis_index("c")
        b0 = wid * BPW

        pltpu.sync_copy(idx_hbm.at[pl.ds(b0 * _TP, BPW * _TP)], idxv)
        pltpu.sync_copy(zrow_hbm, buf0.at[pl.ds(0, 1)])
        pltpu.sync_copy(zrow_hbm, buf1.at[pl.ds(0, 1)])

        bufs = (buf0, buf1)
        gsems = (g0, g1)
        wsems = (w0, w1)

        def gather(j, p):
            return pltpu.async_copy(
                tab_hbm.at[idxv.at[pl.ds(j * _TP, T - 1)]],
                bufs[p].at[pl.ds(1, T - 1)],
                gsems[p])

        gh = {0: gather(0, 0)}
        wh = {}
        for j in range(BPW):
            p = j & 1
            gh[j].wait()
            if j + 1 < BPW:
                if j >= 1:
                    wh[j - 1].wait()
                gh[j + 1] = gather(j + 1, 1 - p)
            wh[j] = pltpu.async_copy(
                bufs[p], out_hbm.at[pl.ds((b0 + j) * T, T)], wsems[p])
        wh[BPW - 2].wait()
        wh[BPW - 1].wait()

    return run


def kernel(idx, log_probs):
    B, T = idx.shape
    V, D = log_probs.shape
    assert B % _NW == 0, (B, _NW)
    assert T - 1 <= _TP
    idx_pad = jnp.zeros((B, _TP), jnp.int32)
    idx_pad = idx_pad.at[:, : T - 1].set(idx[:, : T - 1].astype(jnp.int32))
    zrow = jnp.zeros((1, D), log_probs.dtype)
    out = _build(B, T, V, D, log_probs.dtype)(
        idx_pad.reshape(-1), log_probs, zrow)
    return out.reshape(B, T, D)
